# Initial kernel scaffold; baseline (speedup 1.0000x reference)
import jax
import jax.numpy as jnp
from jax.experimental import pallas as pl


def kernel(indices, weights):
    raise NotImplementedError("write your pallas kernel here")


# speedup vs baseline: 2.7113x; 2.7113x over previous
import functools

import jax
import jax.numpy as jnp
from jax import lax
from jax.experimental import pallas as pl
from jax.experimental.pallas import tpu as pltpu
from jax.experimental.pallas import tpu_sc as plsc

NUM_CORES = 2
NUM_SUBCORES = 16
NW = NUM_CORES * NUM_SUBCORES
D = 128
CHUNK = 128


def _make_gather(B, V):
    b_per_w = B // NW
    n_chunks = b_per_w // CHUNK
    mesh = plsc.VectorSubcoreMesh(core_axis_name="c", subcore_axis_name="s")

    @functools.partial(
        pl.kernel,
        out_type=jax.ShapeDtypeStruct((B, D), jnp.float32),
        mesh=mesh,
        scratch_types=[
            pltpu.VMEM((CHUNK,), jnp.int32),
            pltpu.VMEM((CHUNK, D), jnp.float32),
            pltpu.SemaphoreType.DMA,
        ],
    )
    def gather_kernel(idx_hbm, table_hbm, out_hbm, idx_v, rows_v, sem):
        wid = lax.axis_index("s") * NUM_CORES + lax.axis_index("c")
        base = wid * b_per_w

        def step(c, carry):
            off = base + c * CHUNK
            pltpu.sync_copy(idx_hbm.at[pl.ds(off, CHUNK)], idx_v)
            pltpu.async_copy(table_hbm.at[idx_v], rows_v, sem).wait()
            pltpu.sync_copy(rows_v, out_hbm.at[pl.ds(off, CHUNK)])
            return carry

        lax.fori_loop(0, n_chunks, step, 0)

    return gather_kernel


def kernel(indices, weights):
    bsz, seq_len = indices.shape
    B = bsz * seq_len
    flat = indices.reshape(-1).astype(jnp.int32)
    out = _make_gather(B, weights.shape[0])(flat, weights.astype(jnp.float32))
    return out.reshape(bsz, seq_len, D)


# speedup vs baseline: 3.0147x; 1.1119x over previous
import functools

import jax
import jax.numpy as jnp
from jax import lax
from jax.experimental import pallas as pl
from jax.experimental.pallas import tpu as pltpu
from jax.experimental.pallas import tpu_sc as plsc

NUM_CORES = 2
NUM_SUBCORES = 16
NW = NUM_CORES * NUM_SUBCORES
D = 128
CHUNK = 128
NB = 4


def _make_gather(B, V):
    b_per_w = B // NW
    n_chunks = b_per_w // CHUNK
    n_groups = n_chunks // NB
    mesh = plsc.VectorSubcoreMesh(core_axis_name="c", subcore_axis_name="s")

    @functools.partial(
        pl.kernel,
        out_type=jax.ShapeDtypeStruct((B, D), jnp.float32),
        mesh=mesh,
        scratch_types=[
            pltpu.VMEM((n_chunks, CHUNK), jnp.int32),
            pltpu.VMEM((NB, CHUNK, D), jnp.float32),
            pltpu.SemaphoreType.DMA,
            [pltpu.SemaphoreType.DMA] * NB,
        ],
    )
    def gather_kernel(idx_hbm, table_hbm, out_hbm, idx_v, rows_v, gsem, ssems):
        wid = lax.axis_index("s") * NUM_CORES + lax.axis_index("c")
        base = wid * b_per_w

        pltpu.sync_copy(idx_hbm.at[pl.ds(wid * n_chunks, n_chunks)], idx_v)

        def group(g, carry):
            gathers = []
            for b in range(NB):
                c = g * NB + b

                @pl.when(g > 0)
                def _wait_store(b=b):
                    pltpu.make_async_copy(
                        rows_v.at[b], out_hbm.at[pl.ds(base, CHUNK)], ssems[b]
                    ).wait()

                gathers.append(
                    pltpu.async_copy(table_hbm.at[idx_v.at[c]], rows_v.at[b], gsem)
                )
            for b in range(NB):
                c = g * NB + b
                gathers[b].wait()
                pltpu.async_copy(
                    rows_v.at[b],
                    out_hbm.at[pl.ds(base + c * CHUNK, CHUNK)],
                    ssems[b],
                )
            return carry

        lax.fori_loop(0, n_groups, group, 0)
        for b in range(NB):
            pltpu.make_async_copy(
                rows_v.at[b], out_hbm.at[pl.ds(base, CHUNK)], ssems[b]
            ).wait()

    return gather_kernel


def kernel(indices, weights):
    bsz, seq_len = indices.shape
    B = bsz * seq_len
    flat = indices.reshape(B // CHUNK, CHUNK).astype(jnp.int32)
    out = _make_gather(B, weights.shape[0])(flat, weights.astype(jnp.float32))
    return out.reshape(bsz, seq_len, D)


# speedup vs baseline: 3.9415x; 1.3074x over previous
import functools

import jax
import jax.numpy as jnp
from jax import lax
from jax.experimental import pallas as pl
from jax.experimental.pallas import tpu as pltpu
from jax.experimental.pallas import tpu_sc as plsc

NUM_CORES = 2
NUM_SUBCORES = 16
NW = NUM_CORES * NUM_SUBCORES
D = 128
CHUNK = 128
NB = 4


def _make_gather(B, V):
    b_per_w = B // NW
    n_chunks = b_per_w // CHUNK
    n_groups = n_chunks // NB
    mesh = plsc.VectorSubcoreMesh(core_axis_name="c", subcore_axis_name="s")

    @functools.partial(
        pl.kernel,
        out_type=jax.ShapeDtypeStruct((B, D), jnp.float32),
        mesh=mesh,
        scratch_types=[
            pltpu.VMEM((n_chunks, CHUNK), jnp.int32),
            pltpu.VMEM((NB, CHUNK, D), jnp.float32),
            pltpu.VMEM_SHARED((V, D), jnp.float32),
            pltpu.SemaphoreType.DMA,
            [pltpu.SemaphoreType.DMA] * NB,
        ],
    )
    def gather_kernel(idx_hbm, table_hbm, out_hbm, idx_v, rows_v, table_sp,
                      gsem, ssems):
        wid = lax.axis_index("s") * NUM_CORES + lax.axis_index("c")
        base = wid * b_per_w

        @pl.when(lax.axis_index("s") == 0)
        def _stage_table():
            pltpu.sync_copy(table_hbm, table_sp)

        pltpu.sync_copy(idx_hbm.at[pl.ds(wid * n_chunks, n_chunks)], idx_v)
        plsc.subcore_barrier()

        def group(g, carry):
            gathers = []
            for b in range(NB):
                c = g * NB + b

                @pl.when(g > 0)
                def _wait_store(b=b):
                    pltpu.make_async_copy(
                        rows_v.at[b], out_hbm.at[pl.ds(base, CHUNK)], ssems[b]
                    ).wait()

                gathers.append(
                    pltpu.async_copy(table_sp.at[idx_v.at[c]], rows_v.at[b], gsem)
                )
            for b in range(NB):
                c = g * NB + b
                gathers[b].wait()
                pltpu.async_copy(
                    rows_v.at[b],
                    out_hbm.at[pl.ds(base + c * CHUNK, CHUNK)],
                    ssems[b],
                )
            return carry

        lax.fori_loop(0, n_groups, group, 0)
        for b in range(NB):
            pltpu.make_async_copy(
                rows_v.at[b], out_hbm.at[pl.ds(base, CHUNK)], ssems[b]
            ).wait()

    return gather_kernel


def kernel(indices, weights):
    bsz, seq_len = indices.shape
    B = bsz * seq_len
    flat = indices.reshape(B // CHUNK, CHUNK).astype(jnp.int32)
    out = _make_gather(B, weights.shape[0])(flat, weights.astype(jnp.float32))
    return out.reshape(bsz, seq_len, D)


# speedup vs baseline: 3.9442x; 1.0007x over previous
import functools

import jax
import jax.numpy as jnp
from jax import lax
from jax.experimental import pallas as pl
from jax.experimental.pallas import tpu as pltpu
from jax.experimental.pallas import tpu_sc as plsc

NUM_CORES = 2
NUM_SUBCORES = 16
NW = NUM_CORES * NUM_SUBCORES
D = 128
CHUNK = 128
NB = 5


def _make_gather(B, V):
    b_per_w = B // NW
    n_chunks = b_per_w // CHUNK
    n_groups = n_chunks // NB
    mesh = plsc.VectorSubcoreMesh(core_axis_name="c", subcore_axis_name="s")

    @functools.partial(
        pl.kernel,
        out_type=jax.ShapeDtypeStruct((B, D), jnp.float32),
        mesh=mesh,
        scratch_types=[
            pltpu.VMEM((n_chunks, CHUNK), jnp.int32),
            pltpu.VMEM((NB, CHUNK, D), jnp.float32),
            pltpu.VMEM_SHARED((V, D), jnp.float32),
            pltpu.SemaphoreType.DMA,
            [pltpu.SemaphoreType.DMA] * NB,
        ],
    )
    def gather_kernel(idx_hbm, table_hbm, out_hbm, idx_v, rows_v, table_sp,
                      gsem, ssems):
        wid = lax.axis_index("s") * NUM_CORES + lax.axis_index("c")
        base = wid * b_per_w

        @pl.when(lax.axis_index("s") == 0)
        def _stage_table():
            pltpu.sync_copy(table_hbm, table_sp)

        pltpu.sync_copy(idx_hbm.at[pl.ds(wid * n_chunks, n_chunks)], idx_v)
        plsc.subcore_barrier()

        def group(g, carry):
            gathers = []
            for b in range(NB):
                c = g * NB + b

                @pl.when(g > 0)
                def _wait_store(b=b):
                    pltpu.make_async_copy(
                        rows_v.at[b], out_hbm.at[pl.ds(base, CHUNK)], ssems[b]
                    ).wait()

                gathers.append(
                    pltpu.async_copy(table_sp.at[idx_v.at[c]], rows_v.at[b], gsem)
                )
            for b in range(NB):
                c = g * NB + b
                gathers[b].wait()
                pltpu.async_copy(
                    rows_v.at[b],
                    out_hbm.at[pl.ds(base + c * CHUNK, CHUNK)],
                    ssems[b],
                )
            return carry

        lax.fori_loop(0, n_groups, group, 0)
        for b in range(NB):
            pltpu.make_async_copy(
                rows_v.at[b], out_hbm.at[pl.ds(base, CHUNK)], ssems[b]
            ).wait()

    return gather_kernel


def kernel(indices, weights):
    bsz, seq_len = indices.shape
    B = bsz * seq_len
    flat = indices.reshape(B // CHUNK, CHUNK).astype(jnp.int32)
    out = _make_gather(B, weights.shape[0])(flat, weights.astype(jnp.float32))
    return out.reshape(bsz, seq_len, D)


# speedup vs baseline: 8.1510x; 2.0666x over previous
import functools

import jax
import jax.numpy as jnp
from jax import lax
from jax.experimental import pallas as pl
from jax.experimental.pallas import tpu as pltpu
from jax.experimental.pallas import tpu_sc as plsc

NUM_CORES = 2
NUM_SUBCORES = 16
NW = NUM_CORES * NUM_SUBCORES
D = 128
G = 2
NB = 4


def _make_gather(bsz, seq, V):
    bat_per_w = bsz // NW
    n_chunks = bat_per_w // G
    n_groups = n_chunks // NB
    mesh = plsc.VectorSubcoreMesh(core_axis_name="c", subcore_axis_name="s")

    @functools.partial(
        pl.kernel,
        out_type=jax.ShapeDtypeStruct((bsz, seq, D), jnp.float32),
        mesh=mesh,
        scratch_types=[
            pltpu.VMEM((n_chunks, G * seq), jnp.int32),
            [pltpu.VMEM((G, seq, D), jnp.float32)] * NB,
            pltpu.VMEM_SHARED((V, D), jnp.float32),
            pltpu.SemaphoreType.DMA,
            [pltpu.SemaphoreType.DMA] * NB,
        ],
    )
    def gather_kernel(idx_hbm, table_hbm, out_hbm, idx_v, rows_v, table_sp,
                      gsem, ssems):
        wid = lax.axis_index("s") * NUM_CORES + lax.axis_index("c")
        bat0 = wid * bat_per_w

        @pl.when(lax.axis_index("s") == 0)
        def _stage_table():
            pltpu.sync_copy(table_hbm, table_sp)

        pltpu.sync_copy(idx_hbm.at[pl.ds(wid * n_chunks, n_chunks)], idx_v)
        plsc.subcore_barrier()

        def group(g, carry):
            gathers = []
            for b in range(NB):
                c = g * NB + b

                @pl.when(g > 0)
                def _wait_store(b=b):
                    pltpu.make_async_copy(
                        rows_v[b], out_hbm.at[pl.ds(bat0, G)], ssems[b]
                    ).wait()

                for j in range(G):
                    gathers.append(
                        pltpu.async_copy(
                            table_sp.at[idx_v.at[c, pl.ds(j * seq, seq)]],
                            rows_v[b].at[j],
                            gsem,
                        )
                    )
            for b in range(NB):
                c = g * NB + b
                for j in range(G):
                    gathers[b * G + j].wait()
                pltpu.async_copy(
                    rows_v[b],
                    out_hbm.at[pl.ds(bat0 + c * G, G)],
                    ssems[b],
                )
            return carry

        lax.fori_loop(0, n_groups, group, 0)
        for b in range(NB):
            pltpu.make_async_copy(
                rows_v[b], out_hbm.at[pl.ds(bat0, G)], ssems[b]
            ).wait()

    return gather_kernel


def kernel(indices, weights):
    bsz, seq_len = indices.shape
    idx2d = indices.reshape(bsz // G, G * seq_len).astype(jnp.int32)
    return _make_gather(bsz, seq_len, weights.shape[0])(
        idx2d, weights.astype(jnp.float32))


# speedup vs baseline: 21.1750x; 2.5978x over previous
import functools

import jax
import jax.numpy as jnp
from jax import lax
from jax.experimental import pallas as pl
from jax.experimental.pallas import tpu as pltpu
from jax.experimental.pallas import tpu_sc as plsc

NUM_CORES = 2
NUM_SUBCORES = 16
NW = NUM_CORES * NUM_SUBCORES
D = 128
CH = 128


def _make_gather(bsz, seq, V):
    bat_per_w = bsz // NW
    n_c = bat_per_w // CH
    mesh = plsc.VectorSubcoreMesh(core_axis_name="c", subcore_axis_name="s")

    @functools.partial(
        pl.kernel,
        out_type=jax.ShapeDtypeStruct((seq, bsz, D), jnp.float32),
        mesh=mesh,
        scratch_types=[
            pltpu.VMEM((seq, bat_per_w), jnp.int32),
            pltpu.VMEM((n_c, CH, D), jnp.float32),
            pltpu.VMEM_SHARED((V, D), jnp.float32),
            pltpu.SemaphoreType.DMA,
            [pltpu.SemaphoreType.DMA] * n_c,
        ],
    )
    def gather_kernel(idx_hbm, table_hbm, out_hbm, idx_v, rows_v, table_sp,
                      gsem, ssems):
        wid = lax.axis_index("s") * NUM_CORES + lax.axis_index("c")
        b0 = wid * bat_per_w

        @pl.when(lax.axis_index("s") == 0)
        def _stage_table():
            pltpu.sync_copy(table_hbm, table_sp)

        pltpu.sync_copy(idx_hbm.at[:, pl.ds(b0, bat_per_w)], idx_v)
        plsc.subcore_barrier()

        def per_seq(s, carry):
            gathers = []
            for c in range(n_c):
                @pl.when(s > 0)
                def _wait_store(c=c):
                    pltpu.make_async_copy(
                        rows_v.at[c], out_hbm.at[0, pl.ds(b0, CH)], ssems[c]
                    ).wait()

                gathers.append(
                    pltpu.async_copy(
                        table_sp.at[idx_v.at[s, pl.ds(c * CH, CH)]],
                        rows_v.at[c],
                        gsem,
                    )
                )
            for c in range(n_c):
                gathers[c].wait()
                pltpu.async_copy(
                    rows_v.at[c],
                    out_hbm.at[s, pl.ds(b0 + c * CH, CH)],
                    ssems[c],
                )
            return carry

        lax.fori_loop(0, seq, per_seq, 0)
        for c in range(n_c):
            pltpu.make_async_copy(
                rows_v.at[c], out_hbm.at[0, pl.ds(b0, CH)], ssems[c]
            ).wait()

    return gather_kernel


def kernel(indices, weights):
    bsz, seq_len = indices.shape
    idx_t = jnp.swapaxes(indices, 0, 1).astype(jnp.int32)
    out = _make_gather(bsz, seq_len, weights.shape[0])(
        idx_t, weights.astype(jnp.float32))
    return jnp.swapaxes(out, 0, 1)
